# Initial kernel scaffold; baseline (speedup 1.0000x reference)
import jax
import jax.numpy as jnp
from jax.experimental import pallas as pl


def kernel(X, z, dropout_mask, dense_w, dense_b, up0_w, up0_bn_gamma, up0_bn_beta, up0_bn_mean, up0_bn_var, up0_prelu, up1_w, up1_bn_gamma, up1_bn_beta, up1_bn_mean, up1_bn_var, up1_prelu, up2_w, up2_bn_gamma, up2_bn_beta, up2_bn_mean, up2_bn_var, up2_prelu, proj_w, proj_b, gru_wih, gru_whh, gru_bih, gru_bhh, out_w, out_b):
    raise NotImplementedError("write your pallas kernel here")


# speedup vs baseline: 35.5682x; 35.5682x over previous
import numpy as np
import jax
import jax.numpy as jnp
from jax.experimental import pallas as pl
from jax.experimental.pallas import tpu as pltpu

_NL_REAL = 14
_NL = 16
_NZ = 8
_NC = 4
_CH = 8
_GH = 32
_LOWF = 64
_L0 = 2
_COLS = 128
_STEPS = 14
_GXW = _STEPS * 128
_KIN = 192
_OUTW = _NL_REAL * _NC
_GWROWS = (_STEPS + 1) * _GH + 8
_BT = 256
_BN_EPS = 1e-5


def _body(z_ref, x_ref, m_ref, dw_ref, uw_ref, ua_ref, giw_ref, gw_ref,
          r_ref, o_ref, gx_ref):
    f32 = jnp.float32

    h = (jnp.dot(z_ref[...], dw_ref[0:_NZ, :], preferred_element_type=f32)
         + dw_ref[_NZ:_NZ + 1, :])

    for i in range(3):
        y = jnp.dot(h, uw_ref[i], preferred_element_type=f32) + ua_ref[i:i + 1, :]
        h = jnp.where(y > 0.0, y, ua_ref[4 + i:5 + i, :] * y)

    m56 = jnp.dot(m_ref[...], r_ref[...], preferred_element_type=f32)
    xm = x_ref[...] * m56
    ones = jnp.ones((_BT, 8), f32)
    hx = jnp.concatenate([h, xm, ones], axis=1)

    gx_ref[...] = jnp.dot(hx, giw_ref[...], preferred_element_type=f32)

    bias = gw_ref[_GWROWS - 8:_GWROWS - 7, :]
    hprev = jnp.zeros((_BT, _GH), f32)
    acc = jnp.zeros((_BT, _OUTW), f32)
    for t in range(_STEPS + 1):
        if t == 0:
            s = jnp.broadcast_to(bias, (_BT, 256))
        else:
            s = (jnp.dot(hprev, gw_ref[t * _GH:(t + 1) * _GH, :],
                         preferred_element_type=f32) + bias)
            acc = acc + s[:, 128:128 + _OUTW]
        if t < _STEPS:
            gx = gx_ref[:, t * 128:(t + 1) * 128]
            gh = s[:, 0:128]
            ru = jax.nn.sigmoid(gx + gh)
            r = ru[:, 0:_GH]
            u = ru[:, _GH:2 * _GH]
            n = jnp.tanh(gx[:, 2 * _GH:3 * _GH] + r * gh[:, 2 * _GH:3 * _GH])
            hprev = (1.0 - u) * n + u * hprev
    o_ref[...] = acc + ua_ref[3:4, 0:_OUTW]


def kernel(X, z, dropout_mask, dense_w, dense_b,
           up0_w, up0_bn_gamma, up0_bn_beta, up0_bn_mean, up0_bn_var, up0_prelu,
           up1_w, up1_bn_gamma, up1_bn_beta, up1_bn_mean, up1_bn_var, up1_prelu,
           up2_w, up2_bn_gamma, up2_bn_beta, up2_bn_mean, up2_bn_var, up2_prelu,
           proj_w, proj_b, gru_wih, gru_whh, gru_bih, gru_bhh, out_w, out_b):
    f32 = jnp.float32
    B = X.shape[0]
    nb = -(-B // _BT)
    Bp = nb * _BT

    xr = jnp.pad(X.astype(f32).reshape(B, _NL_REAL * _NC), ((0, Bp - B), (0, 0)))
    mr = jnp.pad(dropout_mask.astype(f32), ((0, Bp - B), (0, 0)))
    zr = jnp.pad(z.astype(f32), ((0, Bp - B), (0, 0)))

    dwt = jnp.transpose(dense_w.astype(f32).T.reshape(_NZ, _LOWF, _L0),
                        (0, 2, 1)).reshape(_NZ, _COLS)
    dbt = dense_b.astype(f32).reshape(_LOWF, _L0).T.reshape(_COLS)
    dw = jnp.concatenate([dwt, dbt[None], jnp.zeros((7, _COLS), f32)], axis=0)

    uws, shifts, alphas = [], [], []
    for w, g, bt, mu, var, al, l_in in (
            (up0_w, up0_bn_gamma, up0_bn_beta, up0_bn_mean, up0_bn_var, up0_prelu, _L0),
            (up1_w, up1_bn_gamma, up1_bn_beta, up1_bn_mean, up1_bn_var, up1_prelu, 2 * _L0),
            (up2_w, up2_bn_gamma, up2_bn_beta, up2_bn_mean, up2_bn_var, up2_prelu, 4 * _L0)):
        sc = g.astype(f32) / jnp.sqrt(var.astype(f32) + _BN_EPS)
        wf = jnp.concatenate([w.astype(f32)[:, :, 0], w.astype(f32)[:, :, 1]],
                             axis=1) * jnp.tile(sc, 2)[None, :]
        uws.append(jnp.kron(jnp.eye(l_in, dtype=f32), wf))
        shifts.append(jnp.tile(bt.astype(f32) - mu.astype(f32) * sc, 2 * l_in))
        alphas.append(jnp.broadcast_to(al.astype(f32)[0], (_COLS,)))
    uw = jnp.stack(uws)
    ua = jnp.stack(shifts
                   + [jnp.pad(jnp.tile(out_b.astype(f32), _NL_REAL),
                              (0, _COLS - _OUTW))]
                   + alphas + [jnp.zeros((_COLS,), f32)])

    wih = gru_wih.astype(f32)
    wih_h = wih[:, :_CH].T
    wxp = (wih[:, _CH:] @ proj_w.astype(f32)[:, :, 0]).T
    b_gx = gru_bih.astype(f32) + wih[:, _CH:] @ proj_b.astype(f32)
    pad96 = lambda a: jnp.pad(a, ((0, 0), (0, 128 - 3 * _GH)))
    w_h = jnp.kron(jnp.eye(_NL, _STEPS, dtype=f32), pad96(wih_h))
    w_x = jnp.kron(jnp.eye(_STEPS, _STEPS, 1, dtype=f32), pad96(wxp))[:_OUTW]
    brow = jnp.tile(jnp.pad(b_gx, (0, 128 - 3 * _GH)), _STEPS)
    giw = jnp.concatenate([w_h, w_x, brow[None],
                           jnp.zeros((_KIN - 185, _GXW), f32)], axis=0)

    whp = jnp.pad(gru_whh.astype(f32).T, ((0, 0), (0, 256 - 3 * _GH)))
    gw = jnp.tile(whp, (_STEPS + 1, 1))
    osh = jnp.kron(jnp.eye(_STEPS + 1, _STEPS, -1, dtype=f32),
                   out_w.astype(f32)[:, :, 0].T)
    gw = gw.at[:, 128:128 + _OUTW].set(osh)
    gbias = jnp.pad(gru_bhh.astype(f32), (0, 256 - 3 * _GH))
    gw = jnp.concatenate([gw, gbias[None], jnp.zeros((7, 256), f32)], axis=0)

    rmat = jnp.kron(jnp.eye(_NL, _STEPS, -1, dtype=f32),
                    jnp.ones((1, _NC), f32))

    grid_spec = pltpu.PrefetchScalarGridSpec(
        num_scalar_prefetch=0,
        grid=(nb,),
        in_specs=[
            pl.BlockSpec((_BT, _NZ), lambda i: (i, 0)),
            pl.BlockSpec((_BT, _NL_REAL * _NC), lambda i: (i, 0)),
            pl.BlockSpec((_BT, _NL), lambda i: (i, 0)),
            pl.BlockSpec((16, _COLS), lambda i: (0, 0)),
            pl.BlockSpec((3, _COLS, _COLS), lambda i: (0, 0, 0)),
            pl.BlockSpec((8, _COLS), lambda i: (0, 0)),
            pl.BlockSpec((_KIN, _GXW), lambda i: (0, 0)),
            pl.BlockSpec((_GWROWS, 256), lambda i: (0, 0)),
            pl.BlockSpec((_NL, _OUTW), lambda i: (0, 0)),
        ],
        out_specs=pl.BlockSpec((_BT, _OUTW), lambda i: (i, 0)),
        scratch_shapes=[pltpu.VMEM((_BT, _GXW), jnp.float32)],
    )

    out = pl.pallas_call(
        _body,
        out_shape=jax.ShapeDtypeStruct((Bp, _OUTW), jnp.float32),
        grid_spec=grid_spec,
        compiler_params=pltpu.CompilerParams(dimension_semantics=("parallel",)),
    )(zr, xr, mr, dw, uw, ua, giw, gw, rmat)

    return out[:B].reshape(B, _NL_REAL, _NC)


# speedup vs baseline: 65.9295x; 1.8536x over previous
import numpy as np
import jax
import jax.numpy as jnp
from jax.experimental import pallas as pl
from jax.experimental.pallas import tpu as pltpu

_NL_REAL = 14
_NL = 16
_NZ = 8
_NC = 4
_CH = 8
_GH = 32
_LOWF = 64
_L0 = 2
_COLS = 128
_STEPS = 14
_GXH = _STEPS * 128
_KIN = 192
_OUTW = _NL_REAL * _NC
_SB = 256
_BT = 256
_BN_EPS = 1e-5


def _body(z_ref, x_ref, m_ref, dw_ref, uw_ref, ua_ref, giw_ref, gw_ref,
          r_ref, o_ref, gx_ref):
    f32 = jnp.float32

    def bcast(col):
        return jnp.broadcast_to(col, (col.shape[0], _BT))

    h = (jnp.dot(dw_ref[:, 0:_NZ], z_ref[...], preferred_element_type=f32)
         + bcast(dw_ref[:, _NZ:_NZ + 1]))

    for i in range(3):
        y = (jnp.dot(uw_ref[i], h, preferred_element_type=f32)
             + bcast(ua_ref[:, i:i + 1]))
        h = jnp.where(y > 0.0, y, bcast(ua_ref[:, 4 + i:5 + i]) * y)

    m56 = jnp.dot(r_ref[...], m_ref[...], preferred_element_type=f32)
    xm = x_ref[...] * m56
    ones = jnp.ones((8, _BT), f32)
    hx = jnp.concatenate([h, xm, ones], axis=0)

    gx_ref[...] = jnp.dot(giw_ref[...], hx, preferred_element_type=f32)

    biasb = bcast(gw_ref[0:_SB, _GH:_GH + 1])
    outb = bcast(ua_ref[0:_OUTW, 3:4])
    hprev = jnp.zeros((_GH, _BT), f32)
    acc = outb
    for t in range(_STEPS + 1):
        if t == 0:
            s = biasb
        else:
            s = (jnp.dot(gw_ref[t * _SB:(t + 1) * _SB, 0:_GH], hprev,
                         preferred_element_type=f32) + biasb)
            acc = acc + s[128:128 + _OUTW, :]
        if t < _STEPS:
            gx = gx_ref[t * 128:t * 128 + 96, :]
            ru = jax.nn.sigmoid(gx[0:2 * _GH, :] + s[0:2 * _GH, :])
            u = ru[_GH:2 * _GH, :]
            n = jnp.tanh(gx[2 * _GH:3 * _GH, :]
                         + ru[0:_GH, :] * s[2 * _GH:3 * _GH, :])
            hprev = n + u * (hprev - n)
    o_ref[...] = acc


def kernel(X, z, dropout_mask, dense_w, dense_b,
           up0_w, up0_bn_gamma, up0_bn_beta, up0_bn_mean, up0_bn_var, up0_prelu,
           up1_w, up1_bn_gamma, up1_bn_beta, up1_bn_mean, up1_bn_var, up1_prelu,
           up2_w, up2_bn_gamma, up2_bn_beta, up2_bn_mean, up2_bn_var, up2_prelu,
           proj_w, proj_b, gru_wih, gru_whh, gru_bih, gru_bhh, out_w, out_b):
    f32 = jnp.float32
    B = X.shape[0]
    nb = -(-B // _BT)
    Bp = nb * _BT

    pad = lambda a: jnp.pad(a, ((0, 0), (0, Bp - B)))
    xr = pad(X.astype(f32).reshape(B, _NL_REAL * _NC).T)
    mr = pad(dropout_mask.astype(f32).T)
    zr = pad(z.astype(f32).T)

    dwt = jnp.transpose(dense_w.astype(f32).T.reshape(_NZ, _LOWF, _L0),
                        (0, 2, 1)).reshape(_NZ, _COLS)
    dbt = dense_b.astype(f32).reshape(_LOWF, _L0).T.reshape(_COLS)
    dw = jnp.concatenate([dwt.T, dbt[:, None],
                          jnp.zeros((_COLS, 7), f32)], axis=1)

    uws, cols = [], []
    for w, g, bt, mu, var, al, l_in in (
            (up0_w, up0_bn_gamma, up0_bn_beta, up0_bn_mean, up0_bn_var, up0_prelu, _L0),
            (up1_w, up1_bn_gamma, up1_bn_beta, up1_bn_mean, up1_bn_var, up1_prelu, 2 * _L0),
            (up2_w, up2_bn_gamma, up2_bn_beta, up2_bn_mean, up2_bn_var, up2_prelu, 4 * _L0)):
        sc = g.astype(f32) / jnp.sqrt(var.astype(f32) + _BN_EPS)
        wf = jnp.concatenate([w.astype(f32)[:, :, 0], w.astype(f32)[:, :, 1]],
                             axis=1) * jnp.tile(sc, 2)[None, :]
        uws.append(jnp.kron(jnp.eye(l_in, dtype=f32), wf.T))
        cols.append(jnp.tile(bt.astype(f32) - mu.astype(f32) * sc, 2 * l_in))
    uw = jnp.stack(uws)
    alphas = [jnp.broadcast_to(a.astype(f32)[0], (_COLS,))
              for a in (up0_prelu, up1_prelu, up2_prelu)]
    ua = jnp.stack(cols
                   + [jnp.pad(jnp.tile(out_b.astype(f32), _NL_REAL),
                              (0, _COLS - _OUTW))]
                   + alphas + [jnp.zeros((_COLS,), f32)], axis=1)

    wih = gru_wih.astype(f32)
    wih_h = wih[:, :_CH]
    wxp = wih[:, _CH:] @ proj_w.astype(f32)[:, :, 0]
    b_gx = gru_bih.astype(f32) + wih[:, _CH:] @ proj_b.astype(f32)
    pad96 = lambda a: jnp.pad(a, ((0, 128 - 3 * _GH), (0, 0)))
    w_h = jnp.kron(jnp.eye(_STEPS, _NL, dtype=f32), pad96(wih_h))
    w_x = jnp.kron(jnp.eye(_STEPS, _STEPS, -1, dtype=f32), pad96(wxp))
    w_x = w_x[:, :_OUTW]
    brow = jnp.tile(jnp.pad(b_gx, (0, 128 - 3 * _GH))[:, None], (_STEPS, 8))
    giw = jnp.concatenate([w_h, w_x, brow / 8.0], axis=1)

    whp = jnp.pad(gru_whh.astype(f32), ((0, _SB - 3 * _GH), (0, 0)))
    gw3 = jnp.tile(whp, (_STEPS + 1, 1)).reshape(_STEPS + 1, _SB, _GH)
    ow = out_w.astype(f32)[:, :, 0]
    for t in range(1, _STEPS + 1):
        gw3 = gw3.at[t, 128 + _NC * (t - 1):128 + _NC * t, :].set(ow)
    gw = gw3.reshape((_STEPS + 1) * _SB, _GH)
    gbias = jnp.pad(gru_bhh.astype(f32), (0, _SB - 3 * _GH))
    gw = jnp.concatenate(
        [gw, jnp.tile(gbias[:, None], (_STEPS + 1, 8))], axis=1)

    rmat = jnp.kron(jnp.eye(_STEPS, _NL, 1, dtype=f32),
                    jnp.ones((_NC, 1), f32))

    grid_spec = pltpu.PrefetchScalarGridSpec(
        num_scalar_prefetch=0,
        grid=(nb,),
        in_specs=[
            pl.BlockSpec((_NZ, _BT), lambda i: (0, i)),
            pl.BlockSpec((_NL_REAL * _NC, _BT), lambda i: (0, i)),
            pl.BlockSpec((_NL, _BT), lambda i: (0, i)),
            pl.BlockSpec((_COLS, 16), lambda i: (0, 0)),
            pl.BlockSpec((3, _COLS, _COLS), lambda i: (0, 0, 0)),
            pl.BlockSpec((_COLS, 8), lambda i: (0, 0)),
            pl.BlockSpec((_GXH, _KIN), lambda i: (0, 0)),
            pl.BlockSpec(((_STEPS + 1) * _SB, 40), lambda i: (0, 0)),
            pl.BlockSpec((_OUTW, _NL), lambda i: (0, 0)),
        ],
        out_specs=pl.BlockSpec((_OUTW, _BT), lambda i: (0, i)),
        scratch_shapes=[pltpu.VMEM((_GXH, _BT), jnp.float32)],
    )

    out = pl.pallas_call(
        _body,
        out_shape=jax.ShapeDtypeStruct((_OUTW, Bp), jnp.float32),
        grid_spec=grid_spec,
        compiler_params=pltpu.CompilerParams(dimension_semantics=("parallel",)),
    )(zr, xr, mr, dw, uw, ua, giw, gw, rmat)

    return out[:, :B].T.reshape(B, _NL_REAL, _NC)


# speedup vs baseline: 105.8116x; 1.6049x over previous
import numpy as np
import jax
import jax.numpy as jnp
from jax.experimental import pallas as pl
from jax.experimental.pallas import tpu as pltpu

_NL_REAL = 14
_NL = 16
_NZ = 8
_NC = 4
_CH = 8
_GH = 32
_LOWF = 64
_L0 = 2
_COLS = 128
_STEPS = 14
_GXH = _STEPS * 128
_KIN = 192
_OUTW = _NL_REAL * _NC
_SB = 256
_BT = 512
_NCH = 2
_CW = _BT // _NCH
_BN_EPS = 1e-5


def _body(z_ref, x_ref, m_ref, dw_ref, uw_ref, ua_ref, giw_ref, gw_ref,
          r_ref, o_ref, gx_ref):
    f32 = jnp.float32

    def bcast(col):
        return jnp.broadcast_to(col, (col.shape[0], _BT))

    h = (jnp.dot(dw_ref[:, 0:_NZ], z_ref[...], preferred_element_type=f32)
         + bcast(dw_ref[:, _NZ:_NZ + 1]))

    for i in range(3):
        y = (jnp.dot(uw_ref[i], h, preferred_element_type=f32)
             + bcast(ua_ref[:, i:i + 1]))
        h = jnp.where(y > 0.0, y, bcast(ua_ref[:, 4 + i:5 + i]) * y)

    m56 = jnp.dot(r_ref[...], m_ref[...], preferred_element_type=f32)
    xm = x_ref[...] * m56
    ones = jnp.ones((8, _BT), f32)
    hx = jnp.concatenate([h, xm, ones], axis=0)

    gx_ref[...] = jnp.dot(giw_ref[...], hx, preferred_element_type=f32)

    biasc = jnp.broadcast_to(gw_ref[0:_SB, _GH:_GH + 1], (_SB, _CW))
    outc = jnp.broadcast_to(ua_ref[0:_OUTW, 3:4], (_OUTW, _CW))
    hprev = [jnp.zeros((_GH, _CW), f32) for _ in range(_NCH)]
    acc = [outc for _ in range(_NCH)]
    for t in range(_STEPS + 1):
        for c in range(_NCH):
            lo = c * _CW
            if t == 0:
                s = biasc
            else:
                s = (jnp.dot(gw_ref[t * _SB:(t + 1) * _SB, 0:_GH], hprev[c],
                             preferred_element_type=f32) + biasc)
                acc[c] = acc[c] + s[128:128 + _OUTW, :]
            if t < _STEPS:
                gx = gx_ref[t * 128:t * 128 + 96, lo:lo + _CW]
                ru = jax.nn.sigmoid(gx[0:2 * _GH, :] + s[0:2 * _GH, :])
                u = ru[_GH:2 * _GH, :]
                n = jnp.tanh(gx[2 * _GH:3 * _GH, :]
                             + ru[0:_GH, :] * s[2 * _GH:3 * _GH, :])
                hprev[c] = n + u * (hprev[c] - n)
    o_ref[...] = jnp.concatenate(acc, axis=1)


def kernel(X, z, dropout_mask, dense_w, dense_b,
           up0_w, up0_bn_gamma, up0_bn_beta, up0_bn_mean, up0_bn_var, up0_prelu,
           up1_w, up1_bn_gamma, up1_bn_beta, up1_bn_mean, up1_bn_var, up1_prelu,
           up2_w, up2_bn_gamma, up2_bn_beta, up2_bn_mean, up2_bn_var, up2_prelu,
           proj_w, proj_b, gru_wih, gru_whh, gru_bih, gru_bhh, out_w, out_b):
    f32 = jnp.float32
    B = X.shape[0]
    nb = -(-B // _BT)
    Bp = nb * _BT

    pad = lambda a: jnp.pad(a, ((0, 0), (0, Bp - B)))
    xr = pad(X.astype(f32).reshape(B, _NL_REAL * _NC).T)
    mr = pad(dropout_mask.astype(f32).T)
    zr = pad(z.astype(f32).T)

    dwt = jnp.transpose(dense_w.astype(f32).T.reshape(_NZ, _LOWF, _L0),
                        (0, 2, 1)).reshape(_NZ, _COLS)
    dbt = dense_b.astype(f32).reshape(_LOWF, _L0).T.reshape(_COLS)
    dw = jnp.concatenate([dwt.T, dbt[:, None],
                          jnp.zeros((_COLS, 7), f32)], axis=1)

    uws, cols = [], []
    for w, g, bt, mu, var, al, l_in in (
            (up0_w, up0_bn_gamma, up0_bn_beta, up0_bn_mean, up0_bn_var, up0_prelu, _L0),
            (up1_w, up1_bn_gamma, up1_bn_beta, up1_bn_mean, up1_bn_var, up1_prelu, 2 * _L0),
            (up2_w, up2_bn_gamma, up2_bn_beta, up2_bn_mean, up2_bn_var, up2_prelu, 4 * _L0)):
        sc = g.astype(f32) / jnp.sqrt(var.astype(f32) + _BN_EPS)
        wf = jnp.concatenate([w.astype(f32)[:, :, 0], w.astype(f32)[:, :, 1]],
                             axis=1) * jnp.tile(sc, 2)[None, :]
        uws.append(jnp.kron(jnp.eye(l_in, dtype=f32), wf.T))
        cols.append(jnp.tile(bt.astype(f32) - mu.astype(f32) * sc, 2 * l_in))
    uw = jnp.stack(uws)
    alphas = [jnp.broadcast_to(a.astype(f32)[0], (_COLS,))
              for a in (up0_prelu, up1_prelu, up2_prelu)]
    ua = jnp.stack(cols
                   + [jnp.pad(jnp.tile(out_b.astype(f32), _NL_REAL),
                              (0, _COLS - _OUTW))]
                   + alphas + [jnp.zeros((_COLS,), f32)], axis=1)

    wih = gru_wih.astype(f32)
    wih_h = wih[:, :_CH]
    wxp = wih[:, _CH:] @ proj_w.astype(f32)[:, :, 0]
    b_gx = gru_bih.astype(f32) + wih[:, _CH:] @ proj_b.astype(f32)
    pad96 = lambda a: jnp.pad(a, ((0, 128 - 3 * _GH), (0, 0)))
    w_h = jnp.kron(jnp.eye(_STEPS, _NL, dtype=f32), pad96(wih_h))
    w_x = jnp.kron(jnp.eye(_STEPS, _STEPS, -1, dtype=f32), pad96(wxp))
    w_x = w_x[:, :_OUTW]
    brow = jnp.tile(jnp.pad(b_gx, (0, 128 - 3 * _GH))[:, None], (_STEPS, 8))
    giw = jnp.concatenate([w_h, w_x, brow / 8.0], axis=1)

    whp = jnp.pad(gru_whh.astype(f32), ((0, _SB - 3 * _GH), (0, 0)))
    gw3 = jnp.tile(whp, (_STEPS + 1, 1)).reshape(_STEPS + 1, _SB, _GH)
    ow = out_w.astype(f32)[:, :, 0]
    for t in range(1, _STEPS + 1):
        gw3 = gw3.at[t, 128 + _NC * (t - 1):128 + _NC * t, :].set(ow)
    gw = gw3.reshape((_STEPS + 1) * _SB, _GH)
    gbias = jnp.pad(gru_bhh.astype(f32), (0, _SB - 3 * _GH))
    gw = jnp.concatenate(
        [gw, jnp.tile(gbias[:, None], (_STEPS + 1, 8))], axis=1)

    rmat = jnp.kron(jnp.eye(_STEPS, _NL, 1, dtype=f32),
                    jnp.ones((_NC, 1), f32))

    grid_spec = pltpu.PrefetchScalarGridSpec(
        num_scalar_prefetch=0,
        grid=(nb,),
        in_specs=[
            pl.BlockSpec((_NZ, _BT), lambda i: (0, i)),
            pl.BlockSpec((_NL_REAL * _NC, _BT), lambda i: (0, i)),
            pl.BlockSpec((_NL, _BT), lambda i: (0, i)),
            pl.BlockSpec((_COLS, 16), lambda i: (0, 0)),
            pl.BlockSpec((3, _COLS, _COLS), lambda i: (0, 0, 0)),
            pl.BlockSpec((_COLS, 8), lambda i: (0, 0)),
            pl.BlockSpec((_GXH, _KIN), lambda i: (0, 0)),
            pl.BlockSpec(((_STEPS + 1) * _SB, 40), lambda i: (0, 0)),
            pl.BlockSpec((_OUTW, _NL), lambda i: (0, 0)),
        ],
        out_specs=pl.BlockSpec((_OUTW, _BT), lambda i: (0, i)),
        scratch_shapes=[pltpu.VMEM((_GXH, _BT), jnp.float32)],
    )

    out = pl.pallas_call(
        _body,
        out_shape=jax.ShapeDtypeStruct((_OUTW, Bp), jnp.float32),
        grid_spec=grid_spec,
        compiler_params=pltpu.CompilerParams(dimension_semantics=("parallel",)),
    )(zr, xr, mr, dw, uw, ua, giw, gw, rmat)

    return out[:, :B].T.reshape(B, _NL_REAL, _NC)


# speedup vs baseline: 160.9536x; 1.5211x over previous
import numpy as np
import jax
import jax.numpy as jnp
from jax.experimental import pallas as pl
from jax.experimental.pallas import tpu as pltpu

_NL_REAL = 14
_NL = 16
_NZ = 8
_NC = 4
_CH = 8
_GH = 32
_LOWF = 64
_L0 = 2
_COLS = 128
_STEPS = 14
_GXH = _STEPS * 128
_KIN = 192
_OUTW = _NL_REAL * _NC
_SB = 256
_BT = 1024
_NCH = 4
_CW = _BT // _NCH
_BN_EPS = 1e-5


def _body(z_ref, x_ref, m_ref, dw_ref, uw_ref, ua_ref, giw_ref, gw_ref,
          r_ref, o_ref, gx_ref):
    f32 = jnp.float32

    def bcast(col):
        return jnp.broadcast_to(col, (col.shape[0], _BT))

    h = (jnp.dot(dw_ref[:, 0:_NZ], z_ref[...], preferred_element_type=f32)
         + bcast(dw_ref[:, _NZ:_NZ + 1]))

    for i in range(3):
        y = (jnp.dot(uw_ref[i], h, preferred_element_type=f32)
             + bcast(ua_ref[:, i:i + 1]))
        h = jnp.where(y > 0.0, y, bcast(ua_ref[:, 4 + i:5 + i]) * y)

    m56 = jnp.dot(r_ref[...], m_ref[...], preferred_element_type=f32)
    xm = x_ref[...] * m56
    ones = jnp.ones((8, _BT), f32)
    hx = jnp.concatenate([h, xm, ones], axis=0)

    gx_ref[...] = jnp.dot(giw_ref[...], hx, preferred_element_type=f32)

    biasc = jnp.broadcast_to(gw_ref[0:_SB, _GH:_GH + 1], (_SB, _CW))
    outc = jnp.broadcast_to(ua_ref[0:_OUTW, 3:4], (_OUTW, _CW))
    hprev = [jnp.zeros((_GH, _CW), f32) for _ in range(_NCH)]
    acc = [outc for _ in range(_NCH)]
    for t in range(_STEPS + 1):
        for c in range(_NCH):
            lo = c * _CW
            if t == 0:
                s = biasc
            else:
                s = (jnp.dot(gw_ref[t * _SB:(t + 1) * _SB, 0:_GH], hprev[c],
                             preferred_element_type=f32) + biasc)
                acc[c] = acc[c] + s[128:128 + _OUTW, :]
            if t < _STEPS:
                gx = gx_ref[t * 128:t * 128 + 96, lo:lo + _CW]
                ru = jax.nn.sigmoid(gx[0:2 * _GH, :] + s[0:2 * _GH, :])
                u = ru[_GH:2 * _GH, :]
                n = jnp.tanh(gx[2 * _GH:3 * _GH, :]
                             + ru[0:_GH, :] * s[2 * _GH:3 * _GH, :])
                hprev[c] = n + u * (hprev[c] - n)
    o_ref[...] = jnp.concatenate(acc, axis=1)


def kernel(X, z, dropout_mask, dense_w, dense_b,
           up0_w, up0_bn_gamma, up0_bn_beta, up0_bn_mean, up0_bn_var, up0_prelu,
           up1_w, up1_bn_gamma, up1_bn_beta, up1_bn_mean, up1_bn_var, up1_prelu,
           up2_w, up2_bn_gamma, up2_bn_beta, up2_bn_mean, up2_bn_var, up2_prelu,
           proj_w, proj_b, gru_wih, gru_whh, gru_bih, gru_bhh, out_w, out_b):
    f32 = jnp.float32
    B = X.shape[0]
    nb = -(-B // _BT)
    Bp = nb * _BT

    pad = lambda a: jnp.pad(a, ((0, 0), (0, Bp - B)))
    xr = pad(X.astype(f32).reshape(B, _NL_REAL * _NC).T)
    mr = pad(dropout_mask.astype(f32).T)
    zr = pad(z.astype(f32).T)

    dwt = jnp.transpose(dense_w.astype(f32).T.reshape(_NZ, _LOWF, _L0),
                        (0, 2, 1)).reshape(_NZ, _COLS)
    dbt = dense_b.astype(f32).reshape(_LOWF, _L0).T.reshape(_COLS)
    dw = jnp.concatenate([dwt.T, dbt[:, None],
                          jnp.zeros((_COLS, 7), f32)], axis=1)

    uws, cols = [], []
    for w, g, bt, mu, var, al, l_in in (
            (up0_w, up0_bn_gamma, up0_bn_beta, up0_bn_mean, up0_bn_var, up0_prelu, _L0),
            (up1_w, up1_bn_gamma, up1_bn_beta, up1_bn_mean, up1_bn_var, up1_prelu, 2 * _L0),
            (up2_w, up2_bn_gamma, up2_bn_beta, up2_bn_mean, up2_bn_var, up2_prelu, 4 * _L0)):
        sc = g.astype(f32) / jnp.sqrt(var.astype(f32) + _BN_EPS)
        wf = jnp.concatenate([w.astype(f32)[:, :, 0], w.astype(f32)[:, :, 1]],
                             axis=1) * jnp.tile(sc, 2)[None, :]
        uws.append(jnp.kron(jnp.eye(l_in, dtype=f32), wf.T))
        cols.append(jnp.tile(bt.astype(f32) - mu.astype(f32) * sc, 2 * l_in))
    uw = jnp.stack(uws)
    alphas = [jnp.broadcast_to(a.astype(f32)[0], (_COLS,))
              for a in (up0_prelu, up1_prelu, up2_prelu)]
    ua = jnp.stack(cols
                   + [jnp.pad(jnp.tile(out_b.astype(f32), _NL_REAL),
                              (0, _COLS - _OUTW))]
                   + alphas + [jnp.zeros((_COLS,), f32)], axis=1)

    wih = gru_wih.astype(f32)
    wih_h = wih[:, :_CH]
    wxp = wih[:, _CH:] @ proj_w.astype(f32)[:, :, 0]
    b_gx = gru_bih.astype(f32) + wih[:, _CH:] @ proj_b.astype(f32)
    pad96 = lambda a: jnp.pad(a, ((0, 128 - 3 * _GH), (0, 0)))
    w_h = jnp.kron(jnp.eye(_STEPS, _NL, dtype=f32), pad96(wih_h))
    w_x = jnp.kron(jnp.eye(_STEPS, _STEPS, -1, dtype=f32), pad96(wxp))
    w_x = w_x[:, :_OUTW]
    brow = jnp.tile(jnp.pad(b_gx, (0, 128 - 3 * _GH))[:, None], (_STEPS, 8))
    giw = jnp.concatenate([w_h, w_x, brow / 8.0], axis=1)

    whp = jnp.pad(gru_whh.astype(f32), ((0, _SB - 3 * _GH), (0, 0)))
    gw3 = jnp.tile(whp, (_STEPS + 1, 1)).reshape(_STEPS + 1, _SB, _GH)
    ow = out_w.astype(f32)[:, :, 0]
    for t in range(1, _STEPS + 1):
        gw3 = gw3.at[t, 128 + _NC * (t - 1):128 + _NC * t, :].set(ow)
    gw = gw3.reshape((_STEPS + 1) * _SB, _GH)
    gbias = jnp.pad(gru_bhh.astype(f32), (0, _SB - 3 * _GH))
    gw = jnp.concatenate(
        [gw, jnp.tile(gbias[:, None], (_STEPS + 1, 8))], axis=1)

    rmat = jnp.kron(jnp.eye(_STEPS, _NL, 1, dtype=f32),
                    jnp.ones((_NC, 1), f32))

    grid_spec = pltpu.PrefetchScalarGridSpec(
        num_scalar_prefetch=0,
        grid=(nb,),
        in_specs=[
            pl.BlockSpec((_NZ, _BT), lambda i: (0, i)),
            pl.BlockSpec((_NL_REAL * _NC, _BT), lambda i: (0, i)),
            pl.BlockSpec((_NL, _BT), lambda i: (0, i)),
            pl.BlockSpec((_COLS, 16), lambda i: (0, 0)),
            pl.BlockSpec((3, _COLS, _COLS), lambda i: (0, 0, 0)),
            pl.BlockSpec((_COLS, 8), lambda i: (0, 0)),
            pl.BlockSpec((_GXH, _KIN), lambda i: (0, 0)),
            pl.BlockSpec(((_STEPS + 1) * _SB, 40), lambda i: (0, 0)),
            pl.BlockSpec((_OUTW, _NL), lambda i: (0, 0)),
        ],
        out_specs=pl.BlockSpec((_OUTW, _BT), lambda i: (0, i)),
        scratch_shapes=[pltpu.VMEM((_GXH, _BT), jnp.float32)],
    )

    out = pl.pallas_call(
        _body,
        out_shape=jax.ShapeDtypeStruct((_OUTW, Bp), jnp.float32),
        grid_spec=grid_spec,
        compiler_params=pltpu.CompilerParams(dimension_semantics=("parallel",)),
    )(zr, xr, mr, dw, uw, ua, giw, gw, rmat)

    return out[:, :B].T.reshape(B, _NL_REAL, _NC)


# speedup vs baseline: 204.5124x; 1.2706x over previous
import numpy as np
import jax
import jax.numpy as jnp
from jax.experimental import pallas as pl
from jax.experimental.pallas import tpu as pltpu

_NL_REAL = 14
_NL = 16
_NZ = 8
_NC = 4
_CH = 8
_GH = 32
_LOWF = 64
_L0 = 2
_COLS = 128
_STEPS = 14
_GXH = _STEPS * 128
_KIN = 192
_OUTW = _NL_REAL * _NC
_SB = 256
_BT = 2048
_NCH = 8
_CW = _BT // _NCH
_BN_EPS = 1e-5


def _body(z_ref, x_ref, m_ref, dw_ref, uw_ref, ua_ref, giw_ref, gw_ref,
          r_ref, o_ref, gx_ref):
    f32 = jnp.float32

    def bcast(col):
        return jnp.broadcast_to(col, (col.shape[0], _BT))

    h = (jnp.dot(dw_ref[:, 0:_NZ], z_ref[...], preferred_element_type=f32)
         + bcast(dw_ref[:, _NZ:_NZ + 1]))

    for i in range(3):
        y = (jnp.dot(uw_ref[i], h, preferred_element_type=f32)
             + bcast(ua_ref[:, i:i + 1]))
        h = jnp.where(y > 0.0, y, bcast(ua_ref[:, 4 + i:5 + i]) * y)

    m56 = jnp.dot(r_ref[...], m_ref[...], preferred_element_type=f32)
    xm = x_ref[...] * m56
    ones = jnp.ones((8, _BT), f32)
    hx = jnp.concatenate([h, xm, ones], axis=0)

    gx_ref[...] = jnp.dot(giw_ref[...], hx, preferred_element_type=f32)

    biasc = jnp.broadcast_to(gw_ref[0:_SB, _GH:_GH + 1], (_SB, _CW))
    outc = jnp.broadcast_to(ua_ref[0:_OUTW, 3:4], (_OUTW, _CW))
    hprev = [jnp.zeros((_GH, _CW), f32) for _ in range(_NCH)]
    acc = [outc for _ in range(_NCH)]
    for t in range(_STEPS + 1):
        for c in range(_NCH):
            lo = c * _CW
            if t == 0:
                s = biasc
            else:
                s = (jnp.dot(gw_ref[t * _SB:(t + 1) * _SB, 0:_GH], hprev[c],
                             preferred_element_type=f32) + biasc)
                acc[c] = acc[c] + s[128:128 + _OUTW, :]
            if t < _STEPS:
                gx = gx_ref[t * 128:t * 128 + 96, lo:lo + _CW]
                ru = jax.nn.sigmoid(gx[0:2 * _GH, :] + s[0:2 * _GH, :])
                u = ru[_GH:2 * _GH, :]
                n = jnp.tanh(gx[2 * _GH:3 * _GH, :]
                             + ru[0:_GH, :] * s[2 * _GH:3 * _GH, :])
                hprev[c] = n + u * (hprev[c] - n)
    o_ref[...] = jnp.concatenate(acc, axis=1)


def kernel(X, z, dropout_mask, dense_w, dense_b,
           up0_w, up0_bn_gamma, up0_bn_beta, up0_bn_mean, up0_bn_var, up0_prelu,
           up1_w, up1_bn_gamma, up1_bn_beta, up1_bn_mean, up1_bn_var, up1_prelu,
           up2_w, up2_bn_gamma, up2_bn_beta, up2_bn_mean, up2_bn_var, up2_prelu,
           proj_w, proj_b, gru_wih, gru_whh, gru_bih, gru_bhh, out_w, out_b):
    f32 = jnp.float32
    B = X.shape[0]
    nb = -(-B // _BT)
    Bp = nb * _BT

    pad = lambda a: jnp.pad(a, ((0, 0), (0, Bp - B)))
    xr = pad(X.astype(f32).reshape(B, _NL_REAL * _NC).T)
    mr = pad(dropout_mask.astype(f32).T)
    zr = pad(z.astype(f32).T)

    dwt = jnp.transpose(dense_w.astype(f32).T.reshape(_NZ, _LOWF, _L0),
                        (0, 2, 1)).reshape(_NZ, _COLS)
    dbt = dense_b.astype(f32).reshape(_LOWF, _L0).T.reshape(_COLS)
    dw = jnp.concatenate([dwt.T, dbt[:, None],
                          jnp.zeros((_COLS, 7), f32)], axis=1)

    uws, cols = [], []
    for w, g, bt, mu, var, al, l_in in (
            (up0_w, up0_bn_gamma, up0_bn_beta, up0_bn_mean, up0_bn_var, up0_prelu, _L0),
            (up1_w, up1_bn_gamma, up1_bn_beta, up1_bn_mean, up1_bn_var, up1_prelu, 2 * _L0),
            (up2_w, up2_bn_gamma, up2_bn_beta, up2_bn_mean, up2_bn_var, up2_prelu, 4 * _L0)):
        sc = g.astype(f32) / jnp.sqrt(var.astype(f32) + _BN_EPS)
        wf = jnp.concatenate([w.astype(f32)[:, :, 0], w.astype(f32)[:, :, 1]],
                             axis=1) * jnp.tile(sc, 2)[None, :]
        uws.append(jnp.kron(jnp.eye(l_in, dtype=f32), wf.T))
        cols.append(jnp.tile(bt.astype(f32) - mu.astype(f32) * sc, 2 * l_in))
    uw = jnp.stack(uws)
    alphas = [jnp.broadcast_to(a.astype(f32)[0], (_COLS,))
              for a in (up0_prelu, up1_prelu, up2_prelu)]
    ua = jnp.stack(cols
                   + [jnp.pad(jnp.tile(out_b.astype(f32), _NL_REAL),
                              (0, _COLS - _OUTW))]
                   + alphas + [jnp.zeros((_COLS,), f32)], axis=1)

    wih = gru_wih.astype(f32)
    wih_h = wih[:, :_CH]
    wxp = wih[:, _CH:] @ proj_w.astype(f32)[:, :, 0]
    b_gx = gru_bih.astype(f32) + wih[:, _CH:] @ proj_b.astype(f32)
    pad96 = lambda a: jnp.pad(a, ((0, 128 - 3 * _GH), (0, 0)))
    w_h = jnp.kron(jnp.eye(_STEPS, _NL, dtype=f32), pad96(wih_h))
    w_x = jnp.kron(jnp.eye(_STEPS, _STEPS, -1, dtype=f32), pad96(wxp))
    w_x = w_x[:, :_OUTW]
    brow = jnp.tile(jnp.pad(b_gx, (0, 128 - 3 * _GH))[:, None], (_STEPS, 8))
    giw = jnp.concatenate([w_h, w_x, brow / 8.0], axis=1)

    whp = jnp.pad(gru_whh.astype(f32), ((0, _SB - 3 * _GH), (0, 0)))
    gw3 = jnp.tile(whp, (_STEPS + 1, 1)).reshape(_STEPS + 1, _SB, _GH)
    ow = out_w.astype(f32)[:, :, 0]
    for t in range(1, _STEPS + 1):
        gw3 = gw3.at[t, 128 + _NC * (t - 1):128 + _NC * t, :].set(ow)
    gw = gw3.reshape((_STEPS + 1) * _SB, _GH)
    gbias = jnp.pad(gru_bhh.astype(f32), (0, _SB - 3 * _GH))
    gw = jnp.concatenate(
        [gw, jnp.tile(gbias[:, None], (_STEPS + 1, 8))], axis=1)

    rmat = jnp.kron(jnp.eye(_STEPS, _NL, 1, dtype=f32),
                    jnp.ones((_NC, 1), f32))

    grid_spec = pltpu.PrefetchScalarGridSpec(
        num_scalar_prefetch=0,
        grid=(nb,),
        in_specs=[
            pl.BlockSpec((_NZ, _BT), lambda i: (0, i)),
            pl.BlockSpec((_NL_REAL * _NC, _BT), lambda i: (0, i)),
            pl.BlockSpec((_NL, _BT), lambda i: (0, i)),
            pl.BlockSpec((_COLS, 16), lambda i: (0, 0)),
            pl.BlockSpec((3, _COLS, _COLS), lambda i: (0, 0, 0)),
            pl.BlockSpec((_COLS, 8), lambda i: (0, 0)),
            pl.BlockSpec((_GXH, _KIN), lambda i: (0, 0)),
            pl.BlockSpec(((_STEPS + 1) * _SB, 40), lambda i: (0, 0)),
            pl.BlockSpec((_OUTW, _NL), lambda i: (0, 0)),
        ],
        out_specs=pl.BlockSpec((_OUTW, _BT), lambda i: (0, i)),
        scratch_shapes=[pltpu.VMEM((_GXH, _BT), jnp.float32)],
    )

    out = pl.pallas_call(
        _body,
        out_shape=jax.ShapeDtypeStruct((_OUTW, Bp), jnp.float32),
        grid_spec=grid_spec,
        compiler_params=pltpu.CompilerParams(dimension_semantics=("parallel",)),
    )(zr, xr, mr, dw, uw, ua, giw, gw, rmat)

    return out[:, :B].T.reshape(B, _NL_REAL, _NC)


# speedup vs baseline: 253.3946x; 1.2390x over previous
import numpy as np
import jax
import jax.numpy as jnp
from jax.experimental import pallas as pl
from jax.experimental.pallas import tpu as pltpu

_NL_REAL = 14
_NL = 16
_NZ = 8
_NC = 4
_CH = 8
_GH = 32
_LOWF = 64
_L0 = 2
_COLS = 128
_STEPS = 14
_GXH = _STEPS * 96
_KIN = 192
_OUTW = _NL_REAL * _NC
_SB = 160
_BT = 2048
_NCH = 8
_CW = _BT // _NCH
_BN_EPS = 1e-5


def _body(z_ref, x_ref, m_ref, dw_ref, uw_ref, ua_ref, giw_ref, gw_ref,
          r_ref, o_ref, gx_ref):
    f32 = jnp.float32

    def bcast(col):
        return jnp.broadcast_to(col, (col.shape[0], _BT))

    h = (jnp.dot(dw_ref[:, 0:_NZ], z_ref[...], preferred_element_type=f32)
         + bcast(dw_ref[:, _NZ:_NZ + 1]))

    for i in range(3):
        y = (jnp.dot(uw_ref[i], h, preferred_element_type=f32)
             + bcast(ua_ref[:, i:i + 1]))
        h = jnp.where(y > 0.0, y, bcast(ua_ref[:, 4 + i:5 + i]) * y)

    m56 = jnp.dot(r_ref[...], m_ref[...], preferred_element_type=f32)
    xm = x_ref[...] * m56
    ones = jnp.ones((8, _BT), f32)
    hx = jnp.concatenate([h, xm, ones], axis=0)

    gx_ref[...] = jnp.dot(giw_ref[...], hx, preferred_element_type=f32)

    biasc = jnp.broadcast_to(gw_ref[0:_SB, _GH:_GH + 1], (_SB, _CW))
    outc = jnp.broadcast_to(ua_ref[0:_OUTW, 3:4], (_OUTW, _CW))
    hprev = [jnp.zeros((_GH, _CW), f32) for _ in range(_NCH)]
    acc = [outc for _ in range(_NCH)]
    for t in range(_STEPS + 1):
        for c in range(_NCH):
            lo = c * _CW
            if t == 0:
                s = biasc
            else:
                s = (jnp.dot(gw_ref[t * _SB:(t + 1) * _SB, 0:_GH], hprev[c],
                             preferred_element_type=f32) + biasc)
                acc[c] = acc[c] + s[96:96 + _OUTW, :]
            if t < _STEPS:
                gx = gx_ref[t * 96:(t + 1) * 96, lo:lo + _CW]
                ru = jax.nn.sigmoid(gx[0:2 * _GH, :] + s[0:2 * _GH, :])
                u = ru[_GH:2 * _GH, :]
                n = jnp.tanh(gx[2 * _GH:3 * _GH, :]
                             + ru[0:_GH, :] * s[2 * _GH:3 * _GH, :])
                hprev[c] = n + u * (hprev[c] - n)
    o_ref[...] = jnp.concatenate(acc, axis=1)


def kernel(X, z, dropout_mask, dense_w, dense_b,
           up0_w, up0_bn_gamma, up0_bn_beta, up0_bn_mean, up0_bn_var, up0_prelu,
           up1_w, up1_bn_gamma, up1_bn_beta, up1_bn_mean, up1_bn_var, up1_prelu,
           up2_w, up2_bn_gamma, up2_bn_beta, up2_bn_mean, up2_bn_var, up2_prelu,
           proj_w, proj_b, gru_wih, gru_whh, gru_bih, gru_bhh, out_w, out_b):
    f32 = jnp.float32
    B = X.shape[0]
    nb = -(-B // _BT)
    Bp = nb * _BT

    pad = lambda a: jnp.pad(a, ((0, 0), (0, Bp - B)))
    xr = pad(X.astype(f32).reshape(B, _NL_REAL * _NC).T)
    mr = pad(dropout_mask.astype(f32).T)
    zr = pad(z.astype(f32).T)

    dwt = jnp.transpose(dense_w.astype(f32).T.reshape(_NZ, _LOWF, _L0),
                        (0, 2, 1)).reshape(_NZ, _COLS)
    dbt = dense_b.astype(f32).reshape(_LOWF, _L0).T.reshape(_COLS)
    dw = jnp.concatenate([dwt.T, dbt[:, None],
                          jnp.zeros((_COLS, 7), f32)], axis=1)

    uws, cols = [], []
    for w, g, bt, mu, var, al, l_in in (
            (up0_w, up0_bn_gamma, up0_bn_beta, up0_bn_mean, up0_bn_var, up0_prelu, _L0),
            (up1_w, up1_bn_gamma, up1_bn_beta, up1_bn_mean, up1_bn_var, up1_prelu, 2 * _L0),
            (up2_w, up2_bn_gamma, up2_bn_beta, up2_bn_mean, up2_bn_var, up2_prelu, 4 * _L0)):
        sc = g.astype(f32) / jnp.sqrt(var.astype(f32) + _BN_EPS)
        wf = jnp.concatenate([w.astype(f32)[:, :, 0], w.astype(f32)[:, :, 1]],
                             axis=1) * jnp.tile(sc, 2)[None, :]
        uws.append(jnp.kron(jnp.eye(l_in, dtype=f32), wf.T))
        cols.append(jnp.tile(bt.astype(f32) - mu.astype(f32) * sc, 2 * l_in))
    uw = jnp.stack(uws)
    alphas = [jnp.broadcast_to(a.astype(f32)[0], (_COLS,))
              for a in (up0_prelu, up1_prelu, up2_prelu)]
    ua = jnp.stack(cols
                   + [jnp.pad(jnp.tile(out_b.astype(f32), _NL_REAL),
                              (0, _COLS - _OUTW))]
                   + alphas + [jnp.zeros((_COLS,), f32)], axis=1)

    wih = gru_wih.astype(f32)
    wih_h = wih[:, :_CH]
    wxp = wih[:, _CH:] @ proj_w.astype(f32)[:, :, 0]
    b_gx = gru_bih.astype(f32) + wih[:, _CH:] @ proj_b.astype(f32)
    w_h = jnp.kron(jnp.eye(_STEPS, _NL, dtype=f32), wih_h)
    w_x = jnp.kron(jnp.eye(_STEPS, _STEPS, -1, dtype=f32), wxp)
    brow = jnp.tile(b_gx[:, None], (_STEPS, 8))
    giw = jnp.concatenate([w_h, w_x, brow / 8.0], axis=1)

    whp = jnp.pad(gru_whh.astype(f32), ((0, _SB - 3 * _GH), (0, 0)))
    gw3 = jnp.tile(whp, (_STEPS + 1, 1)).reshape(_STEPS + 1, _SB, _GH)
    ow = out_w.astype(f32)[:, :, 0]
    for t in range(1, _STEPS + 1):
        gw3 = gw3.at[t, 96 + _NC * (t - 1):96 + _NC * t, :].set(ow)
    gw = gw3.reshape((_STEPS + 1) * _SB, _GH)
    gbias = jnp.pad(gru_bhh.astype(f32), (0, _SB - 3 * _GH))
    gw = jnp.concatenate(
        [gw, jnp.tile(gbias[:, None], (_STEPS + 1, 8))], axis=1)

    rmat = jnp.kron(jnp.eye(_STEPS, _NL, 1, dtype=f32),
                    jnp.ones((_NC, 1), f32))

    grid_spec = pltpu.PrefetchScalarGridSpec(
        num_scalar_prefetch=0,
        grid=(nb,),
        in_specs=[
            pl.BlockSpec((_NZ, _BT), lambda i: (0, i)),
            pl.BlockSpec((_NL_REAL * _NC, _BT), lambda i: (0, i)),
            pl.BlockSpec((_NL, _BT), lambda i: (0, i)),
            pl.BlockSpec((_COLS, 16), lambda i: (0, 0)),
            pl.BlockSpec((3, _COLS, _COLS), lambda i: (0, 0, 0)),
            pl.BlockSpec((_COLS, 8), lambda i: (0, 0)),
            pl.BlockSpec((_GXH, _KIN), lambda i: (0, 0)),
            pl.BlockSpec(((_STEPS + 1) * _SB, 40), lambda i: (0, 0)),
            pl.BlockSpec((_OUTW, _NL), lambda i: (0, 0)),
        ],
        out_specs=pl.BlockSpec((_OUTW, _BT), lambda i: (0, i)),
        scratch_shapes=[pltpu.VMEM((_GXH, _BT), jnp.float32)],
    )

    out = pl.pallas_call(
        _body,
        out_shape=jax.ShapeDtypeStruct((_OUTW, Bp), jnp.float32),
        grid_spec=grid_spec,
        compiler_params=pltpu.CompilerParams(dimension_semantics=("parallel",)),
    )(zr, xr, mr, dw, uw, ua, giw, gw, rmat)

    return out[:, :B].T.reshape(B, _NL_REAL, _NC)
